# Initial kernel scaffold; baseline (speedup 1.0000x reference)
import jax
import jax.numpy as jnp
from jax.experimental import pallas as pl


def kernel(transitions, type_mask, rules, weights, biases, t_sections, c_sections):
    raise NotImplementedError("write your pallas kernel here")


# speedup vs baseline: 2.1920x; 2.1920x over previous
import jax
import jax.numpy as jnp
import numpy as np
from jax.experimental import pallas as pl
from jax.experimental.pallas import tpu as pltpu


def _tc_body(trans_ref, tmask_ref, rules_ref, weights_ref, biases_ref, out_ref):
    B, N, _, P = trans_ref.shape
    R = tmask_ref.shape[-1]
    Nc = rules_ref.shape[-1]

    trans = trans_ref[...]
    th = trans.reshape(B * N * N, P)

    pgrid = jax.lax.broadcasted_iota(jnp.int32, (P, Nc), 0)
    oh0 = (pgrid == rules_ref[0:1, :]).astype(jnp.float32)
    oh1 = (pgrid == rules_ref[1:2, :]).astype(jnp.float32)

    rm = jnp.max(trans, axis=2).reshape(B * N, P)
    rmg = jnp.dot(rm, oh0, preferred_element_type=jnp.float32, precision=jax.lax.Precision.HIGHEST)
    t1 = jnp.dot(th, oh1, preferred_element_type=jnp.float32, precision=jax.lax.Precision.HIGHEST)
    path = rmg.reshape(B, N, 1, Nc) + t1.reshape(B, N, N, Nc)

    scores = jnp.exp(path).reshape(B * N * N, Nc)

    C = Nc // 3
    cgrid = jax.lax.broadcasted_iota(jnp.int32, (Nc, C), 0)
    igrid = jax.lax.broadcasted_iota(jnp.int32, (Nc, C), 1)
    w = jnp.zeros((Nc, C), jnp.float32)
    for m in range(3):
        w = w + jnp.where(cgrid == 3 * igrid + m, 1.0, 0.0) * weights_ref[m:m + 1, :]
    s16 = jnp.dot(scores, w, preferred_element_type=jnp.float32, precision=jax.lax.Precision.HIGHEST) + biases_ref[0:1, :]

    c16 = jax.lax.broadcasted_iota(jnp.int32, (C, R), 0)
    r16 = jax.lax.broadcasted_iota(jnp.int32, (C, R), 1)
    e0 = (c16 == 2 * r16).astype(jnp.float32)
    e1 = (c16 == 2 * r16 + 1).astype(jnp.float32)
    sel0 = jnp.dot(s16, e0, preferred_element_type=jnp.float32, precision=jax.lax.Precision.HIGHEST).reshape(B, N, N, R)
    sel1 = jnp.dot(s16, e1, preferred_element_type=jnp.float32, precision=jax.lax.Precision.HIGHEST).reshape(B, N, N, R)
    out_ref[...] = jnp.where(tmask_ref[...] == 0, sel0, sel1)


def kernel(transitions, type_mask, rules, weights, biases, t_sections, c_sections):
    B, N, _, _ = transitions.shape
    R = type_mask.shape[-1]
    rules_t = rules.T
    weights_t = weights[:, :, 0].T
    biases_r = biases.reshape(1, -1)
    return pl.pallas_call(
        _tc_body,
        out_shape=jax.ShapeDtypeStruct((B, N, N, R), transitions.dtype),
    )(transitions, type_mask, rules_t, weights_t, biases_r)
